# baseline (speedup 1.0000x reference)
import functools
import math

import jax
import jax.numpy as jnp
from jax import lax
from jax.experimental import pallas as pl

_DN_T = (((1,), (1,)), ((), ()))


def _mlp_kernel(x_ref, phi_ref, w1_ref, b1_ref, w2_ref, b2_ref, w3_ref,
                b3_ref, o_ref):
    w = phi_ref[...] * (2.0 / math.pi)
    wi = w.astype(jnp.int32)
    w2 = w * w
    w3 = w2 * w
    cs = (
        -0.5 * w + w2 - 0.5 * w3,
        -2.5 * w2 + 1.5 * w3,
        0.5 * w + 2.0 * w2 - 1.5 * w3,
        -0.5 * w2 + 0.5 * w3,
    )
    d = []
    for e in range(4):
        de = jnp.zeros_like(w)
        for i in range(4):
            ki = jnp.bitwise_and(wi + (i + 3), 3)
            de = de + jnp.where(ki == e, cs[i], 0.0)
        d.append(de)
    d4 = jnp.concatenate(d, axis=1)

    h = x_ref[...]
    for w_ref, b_ref, act in ((w1_ref, b1_ref, True),
                              (w2_ref, b2_ref, True),
                              (w3_ref, b3_ref, False)):
        acc = jnp.dot(d4, b_ref[...], preferred_element_type=jnp.float32)
        for e in range(4):
            y = lax.dot_general(h, w_ref[e], _DN_T,
                                precision=lax.Precision.DEFAULT,
                                preferred_element_type=jnp.float32)
            acc = acc + d[e] * y
        if act:
            h = jnp.maximum(acc, 0.0)
        else:
            o_ref[...] = acc


@functools.partial(jax.jit, static_argnames=())
def kernel(X, phi, W1, b1, W2, b2, W3, b3):
    B, IN = X.shape
    HID = W1.shape[1]
    OUT = W3.shape[1]
    BT = 256

    phi2 = phi.reshape(B, 1)

    return pl.pallas_call(
        _mlp_kernel,
        grid=(B // BT,),
        in_specs=[
            pl.BlockSpec((BT, IN), lambda i: (i, 0)),
            pl.BlockSpec((BT, 1), lambda i: (i, 0)),
            pl.BlockSpec((4, HID, IN), lambda i: (0, 0, 0)),
            pl.BlockSpec((4, HID), lambda i: (0, 0)),
            pl.BlockSpec((4, HID, HID), lambda i: (0, 0, 0)),
            pl.BlockSpec((4, HID), lambda i: (0, 0)),
            pl.BlockSpec((4, OUT, HID), lambda i: (0, 0, 0)),
            pl.BlockSpec((4, OUT), lambda i: (0, 0)),
        ],
        out_specs=pl.BlockSpec((BT, OUT), lambda i: (i, 0)),
        out_shape=jax.ShapeDtypeStruct((B, OUT), jnp.float32),
    )(X, phi2, W1, b1, W2, b2, W3, b3)


# speedup vs baseline: 1.0077x; 1.0077x over previous
import functools
import math

import jax
import jax.numpy as jnp
from jax import lax
from jax.experimental import pallas as pl
from jax.experimental.pallas import tpu as pltpu

_DN_T = (((1,), (1,)), ((), ()))


def _mlp_kernel(x_ref, phi_ref, w1_ref, b1_ref, w2_ref, b2_ref, w3_ref,
                b3_ref, o_ref):
    w = phi_ref[...] * (2.0 / math.pi)
    wi = w.astype(jnp.int32)
    w2 = w * w
    w3 = w2 * w
    cs = (
        -0.5 * w + w2 - 0.5 * w3,
        -2.5 * w2 + 1.5 * w3,
        0.5 * w + 2.0 * w2 - 1.5 * w3,
        -0.5 * w2 + 0.5 * w3,
    )
    d = []
    for e in range(4):
        de = jnp.zeros_like(w)
        for i in range(4):
            ki = jnp.bitwise_and(wi + (i + 3), 3)
            de = de + jnp.where(ki == e, cs[i], 0.0)
        d.append(de)
    d4 = jnp.concatenate(d, axis=1)

    h = x_ref[...]
    for w_ref, b_ref, act in ((w1_ref, b1_ref, True),
                              (w2_ref, b2_ref, True),
                              (w3_ref, b3_ref, False)):
        acc = jnp.dot(d4, b_ref[...], preferred_element_type=jnp.float32)
        for e in range(4):
            y = lax.dot_general(h, w_ref[e], _DN_T,
                                precision=lax.Precision.DEFAULT,
                                preferred_element_type=jnp.float32)
            acc = acc + d[e] * y
        if act:
            h = jnp.maximum(acc, 0.0)
        else:
            o_ref[...] = acc


@functools.partial(jax.jit, static_argnames=())
def kernel(X, phi, W1, b1, W2, b2, W3, b3):
    B, IN = X.shape
    HID = W1.shape[1]
    OUT = W3.shape[1]
    BT = 256

    phi2 = phi.reshape(B, 1)

    return pl.pallas_call(
        _mlp_kernel,
        grid=(B // BT,),
        in_specs=[
            pl.BlockSpec((BT, IN), lambda i: (i, 0)),
            pl.BlockSpec((BT, 1), lambda i: (i, 0)),
            pl.BlockSpec((4, HID, IN), lambda i: (0, 0, 0)),
            pl.BlockSpec((4, HID), lambda i: (0, 0)),
            pl.BlockSpec((4, HID, HID), lambda i: (0, 0, 0)),
            pl.BlockSpec((4, HID), lambda i: (0, 0)),
            pl.BlockSpec((4, OUT, HID), lambda i: (0, 0, 0)),
            pl.BlockSpec((4, OUT), lambda i: (0, 0)),
        ],
        out_specs=pl.BlockSpec((BT, OUT), lambda i: (i, 0)),
        out_shape=jax.ShapeDtypeStruct((B, OUT), jnp.float32),
        compiler_params=pltpu.CompilerParams(
            dimension_semantics=("parallel",)),
    )(X, phi2, W1, b1, W2, b2, W3, b3)
